# baseline (speedup 1.0000x reference)
import functools

import jax
import jax.numpy as jnp
from jax import lax
from jax.experimental import pallas as pl
from jax.experimental.pallas import tpu as pltpu
from jax.experimental.pallas import tpu_sc as plsc

B = 32
N = 32768
F = 64
K = 1024
L = 16
NV = N // L


def _max_body(x_ref, o_ref):
    m = jnp.max(x_ref[...], axis=1)
    m = jnp.where(m == 0.0, 0.0, m)
    o_ref[...] = m.reshape(o_ref.shape)


def _compute_maxes(x2d):
    rows = B * N
    blk = 16384
    return pl.pallas_call(
        _max_body,
        grid=(rows // blk,),
        in_specs=[pl.BlockSpec((blk, F), lambda i: (i, 0))],
        out_specs=pl.BlockSpec((blk // 128, 128), lambda i: (i, 0)),
        out_shape=jax.ShapeDtypeStruct((rows // 128, 128), jnp.float32),
    )(x2d)


_sc_mesh = plsc.VectorSubcoreMesh(core_axis_name="c", subcore_axis_name="s")


@functools.partial(
    pl.kernel,
    out_type=jax.ShapeDtypeStruct((B, K, F), jnp.float32),
    mesh=_sc_mesh,
    compiler_params=pltpu.CompilerParams(needs_layout_passes=False),
    scratch_types=[
        pltpu.VMEM((N // 128, 128), jnp.float32),
        pltpu.VMEM((4096,), jnp.int32),
        pltpu.VMEM((256,), jnp.int32),
        pltpu.VMEM((K,), jnp.int32),
        pltpu.VMEM((K,), jnp.int32),
        pltpu.VMEM((K,), jnp.int32),
        pltpu.VMEM((K,), jnp.int32),
        pltpu.VMEM((K,), jnp.int32),
        pltpu.SemaphoreType.DMA,
    ],
)
def _sc_topk(maxes_hbm, table_hbm, out_hbm,
             maxv, hist, offs, akd, aidx, bkd, bidx, fidx, sem):
    NR = N // 128
    b = lax.axis_index("s") * 2 + lax.axis_index("c")
    pltpu.sync_copy(maxes_hbm.at[pl.ds(b * NR, NR)], maxv)

    def zf(i, c):
        fidx[pl.ds(i * L, L)] = jnp.zeros((L,), jnp.int32)
        return c
    lax.fori_loop(0, K // L, zf, 0)

    lanes = lax.iota(jnp.int32, 16)
    ones = jnp.ones((L,), jnp.int32)
    zeros = jnp.zeros((L,), jnp.int32)

    def srl(x, s):
        return lax.shift_right_logical(
            x, jnp.full(jnp.shape(x), s, jnp.int32))

    def zero_hist():
        def zb(i, c):
            hist[pl.ds(i * L, L)] = zeros
            return c
        lax.fori_loop(0, 256, zb, 0)

    prefix = jnp.int32(0)
    cnt_before = jnp.int32(0)
    for p in range(4):
        sh = 24 - 8 * p
        zero_hist()

        def acc(i, c, _p=p, _sh=sh, _prefix=prefix):
            for u in range(8):
                if _p == 0:
                    v = maxv[i, pl.ds(u * L, L)]
                    kb = lax.bitcast_convert_type(v, jnp.int32)
                    kd = jnp.where(kb < 0, kb, (~kb) & jnp.int32(0x7FFFFFFF))
                    maxv[i, pl.ds(u * L, L)] = lax.bitcast_convert_type(
                        kd, jnp.float32)
                    d = srl(kd, _sh) & jnp.int32(255)
                    plsc.addupdate_scatter(hist, [lanes * 256 + d], ones)
                else:
                    kd = lax.bitcast_convert_type(
                        maxv[i, pl.ds(u * L, L)], jnp.int32)
                    d = srl(kd, _sh) & jnp.int32(255)
                    m = srl(kd, _sh + 8) == srl(_prefix, _sh + 8)
                    plsc.addupdate_scatter(hist, [lanes * 256 + d], ones,
                                           mask=m)
            return c
        lax.fori_loop(0, NR, acc, 0)

        def chunk(e, carry):
            crun, tdig, cntb, done = carry
            tot = zeros
            for l in range(16):
                tot = tot + hist[pl.ds(l * 256 + e * L, L)]
            cum = plsc.cumsum(tot)
            reached = (crun + cum) >= K
            nbelow = jnp.sum(jnp.where(reached, 0, 1).astype(jnp.int32))
            below = jnp.sum(jnp.where(reached, 0, tot))
            found = nbelow < 16
            upd = jnp.logical_and(done == 0, found)
            tdig = jnp.where(upd, e * L + nbelow, tdig)
            cntb = jnp.where(upd, crun + below, cntb)
            crun = crun + jnp.sum(tot)
            done = jnp.where(upd, jnp.int32(1), done)
            return crun, tdig, cntb, done

        _, tdig, cntb, _ = lax.fori_loop(
            0, 16, chunk,
            (cnt_before, jnp.int32(0), jnp.int32(0), jnp.int32(0)))
        prefix = prefix | (tdig << sh)
        cnt_before = cntb

    T = prefix
    count_lt = cnt_before
    need_eq = K - count_lt
    MIN32 = jnp.int32(-2147483648)
    Tx = T ^ MIN32

    def comp(i, carry):
        off_lt, off_eq = carry
        for u in range(8):
            kd = lax.bitcast_convert_type(maxv[i, pl.ds(u * L, L)], jnp.int32)
            gidx = b * N + i * 128 + u * L + lanes
            m_lt = (kd ^ MIN32) < Tx
            c = plsc.cumsum(m_lt.astype(jnp.int32))
            pos = off_lt + c - 1
            plsc.store_scatter(akd, [pos], kd, mask=m_lt)
            plsc.store_scatter(aidx, [pos], gidx, mask=m_lt)
            m_eq = kd == T
            ceq = plsc.cumsum(m_eq.astype(jnp.int32))
            m_eq = jnp.logical_and(m_eq, (off_eq + ceq) <= need_eq)
            ceq2 = plsc.cumsum(m_eq.astype(jnp.int32))
            peq = count_lt + off_eq + ceq2 - 1
            plsc.store_scatter(fidx, [peq], gidx, mask=m_eq)
            off_lt = off_lt + jnp.sum(m_lt.astype(jnp.int32))
            off_eq = off_eq + jnp.sum(m_eq.astype(jnp.int32))
        return (off_lt, off_eq)

    lax.fori_loop(0, NR, comp, (jnp.int32(0), jnp.int32(0)))

    nv_lt = (count_lt + (L - 1)) // L
    bufs = [(akd, aidx), (bkd, bidx)]
    for p in range(4):
        sh = 8 * p
        skd, sidx = bufs[p % 2]
        dkd, didx = bufs[(p + 1) % 2]
        zero_hist()

        def hacc(i, c, _sh=sh, _skd=skd):
            m = (i * L + lanes) < count_lt
            kv = _skd[pl.ds(i * L, L)]
            d = srl(kv, _sh) & jnp.int32(255)
            plsc.addupdate_scatter(hist, [lanes * 256 + d], ones, mask=m)
            return c
        lax.fori_loop(0, nv_lt, hacc, 0)

        def offb(e, cin):
            tot = zeros
            for l in range(16):
                tot = tot + hist[pl.ds(l * 256 + e * L, L)]
            cum = plsc.cumsum(tot)
            offs[pl.ds(e * L, L)] = cin + cum - tot
            return cin + jnp.sum(tot)
        lax.fori_loop(0, 16, offb, jnp.int32(0))

        def scat(i, c, _p=p, _sh=sh, _skd=skd, _sidx=sidx, _dkd=dkd, _didx=didx):
            m = (i * L + lanes) < count_lt
            kv = _skd[pl.ds(i * L, L)]
            iv = _sidx[pl.ds(i * L, L)]
            d = srl(kv, _sh) & jnp.int32(255)
            base = plsc.load_gather(offs, [d])
            dup, lastm = plsc.scan_count(d, mask=m)
            pos = base + dup - 1
            if _p == 3:
                plsc.store_scatter(fidx, [pos], iv, mask=m)
            else:
                plsc.store_scatter(_dkd, [pos], kv, mask=m)
                plsc.store_scatter(_didx, [pos], iv, mask=m)
            plsc.addupdate_scatter(offs, [d], dup,
                                   mask=jnp.logical_and(lastm, m))
            return c
        lax.fori_loop(0, nv_lt, scat, 0)

    def grow(i, c):
        idxv = fidx[pl.ds(i * L, L)]
        idxv = jnp.clip(idxv, 0, B * N - 1)
        for j in range(L):
            rid = jnp.sum(jnp.where(lanes == j, idxv, 0))
            pltpu.async_copy(table_hbm.at[rid],
                             out_hbm.at[b, i * L + j], sem)
        return c
    lax.fori_loop(0, K // L, grow, 0)

    def drain(i, c):
        pltpu.make_async_copy(table_hbm.at[0], out_hbm.at[b, 0], sem).wait()
        return c
    lax.fori_loop(0, K, drain, 0)


def kernel(output_of_dgcnn_layer):
    x2d = output_of_dgcnn_layer.reshape(B * N, F)
    maxes = _compute_maxes(x2d)
    return _sc_topk(maxes, x2d)


# speedup vs baseline: 1.2222x; 1.2222x over previous
import functools

import jax
import jax.numpy as jnp
from jax import lax
from jax.experimental import pallas as pl
from jax.experimental.pallas import tpu as pltpu
from jax.experimental.pallas import tpu_sc as plsc

B = 32
N = 32768
F = 64
K = 1024
L = 16
NV = N // L


def _max_body(xt_ref, o_ref):
    m = jnp.max(xt_ref[...], axis=1)
    o_ref[...] = jnp.where(m == 0.0, 0.0, m)


def _compute_maxes(xt):
    blk = 1024
    return pl.pallas_call(
        _max_body,
        grid=(N // blk,),
        in_specs=[pl.BlockSpec((B, F, blk), lambda i: (0, 0, i))],
        out_specs=pl.BlockSpec((B, blk), lambda i: (0, i)),
        out_shape=jax.ShapeDtypeStruct((B, N), jnp.float32),
    )(xt)


_sc_mesh = plsc.VectorSubcoreMesh(core_axis_name="c", subcore_axis_name="s")


@functools.partial(
    pl.kernel,
    out_type=jax.ShapeDtypeStruct((B, K, F), jnp.float32),
    mesh=_sc_mesh,
    compiler_params=pltpu.CompilerParams(needs_layout_passes=False,
                                         use_tc_tiling_on_sc=False),
    scratch_types=[
        pltpu.VMEM((N,), jnp.float32),
        pltpu.VMEM((4096,), jnp.int32),
        pltpu.VMEM((256,), jnp.int32),
        pltpu.VMEM((K,), jnp.int32),
        pltpu.VMEM((K,), jnp.int32),
        pltpu.VMEM((K,), jnp.int32),
        pltpu.VMEM((K,), jnp.int32),
        pltpu.VMEM((K,), jnp.int32),
        pltpu.VMEM((K, F), jnp.float32),
        pltpu.SemaphoreType.DMA,
    ],
)
def _sc_topk(maxes_hbm, table_hbm, out_hbm,
             maxv, hist, offs, akd, aidx, bkd, bidx, fidx, rows, sem):
    b = lax.axis_index("s") * 2 + lax.axis_index("c")
    pltpu.sync_copy(maxes_hbm.at[b], maxv)

    def zf(i, c):
        fidx[pl.ds(i * L, L)] = jnp.zeros((L,), jnp.int32)
        return c
    lax.fori_loop(0, K // L, zf, 0)

    lanes = lax.iota(jnp.int32, 16)
    ones = jnp.ones((L,), jnp.int32)
    zeros = jnp.zeros((L,), jnp.int32)

    def srl(x, s):
        return lax.shift_right_logical(
            x, jnp.full(jnp.shape(x), s, jnp.int32))

    def kd_at(i):
        return lax.bitcast_convert_type(maxv[pl.ds(i * L, L)], jnp.int32)

    def zero_hist():
        def zb(i, c):
            hist[pl.ds(i * L, L)] = zeros
            return c
        lax.fori_loop(0, 256, zb, 0)

    UNROLL = 4
    prefix = jnp.int32(0)
    cnt_before = jnp.int32(0)
    for p in range(4):
        sh = 24 - 8 * p
        zero_hist()

        def acc(i, c, _p=p, _sh=sh, _prefix=prefix):
            for u in range(UNROLL):
                base = (i * UNROLL + u) * L
                if _p == 0:
                    v = maxv[pl.ds(base, L)]
                    kb = lax.bitcast_convert_type(v, jnp.int32)
                    kd = jnp.where(kb < 0, kb, (~kb) & jnp.int32(0x7FFFFFFF))
                    maxv[pl.ds(base, L)] = lax.bitcast_convert_type(
                        kd, jnp.float32)
                    d = srl(kd, _sh) & jnp.int32(255)
                    plsc.addupdate_scatter(hist, [lanes * 256 + d], ones)
                else:
                    kd = lax.bitcast_convert_type(
                        maxv[pl.ds(base, L)], jnp.int32)
                    d = srl(kd, _sh) & jnp.int32(255)
                    m = srl(kd, _sh + 8) == srl(_prefix, _sh + 8)
                    plsc.addupdate_scatter(hist, [lanes * 256 + d], ones,
                                           mask=m)
            return c
        lax.fori_loop(0, NV // UNROLL, acc, 0)

        def chunk(e, carry):
            crun, tdig, cntb, done = carry
            tot = zeros
            for l in range(16):
                tot = tot + hist[pl.ds(l * 256 + e * L, L)]
            cum = plsc.cumsum(tot)
            reached = (crun + cum) >= K
            nbelow = jnp.sum(jnp.where(reached, 0, 1).astype(jnp.int32))
            below = jnp.sum(jnp.where(reached, 0, tot))
            found = nbelow < 16
            upd = jnp.logical_and(done == 0, found)
            tdig = jnp.where(upd, e * L + nbelow, tdig)
            cntb = jnp.where(upd, crun + below, cntb)
            crun = crun + jnp.sum(tot)
            done = jnp.where(upd, jnp.int32(1), done)
            return crun, tdig, cntb, done

        _, tdig, cntb, _ = lax.fori_loop(
            0, 16, chunk,
            (cnt_before, jnp.int32(0), jnp.int32(0), jnp.int32(0)))
        prefix = prefix | (tdig << sh)
        cnt_before = cntb

    T = prefix
    count_lt = cnt_before
    need_eq = K - count_lt
    MIN32 = jnp.int32(-2147483648)
    Tx = T ^ MIN32

    def comp(i, carry):
        off_lt, off_eq = carry
        for u in range(UNROLL):
            j = i * UNROLL + u
            kd = kd_at(j)
            gidx = b * N + j * L + lanes
            m_lt = (kd ^ MIN32) < Tx
            c = plsc.cumsum(m_lt.astype(jnp.int32))
            pos = off_lt + c - 1
            plsc.store_scatter(akd, [pos], kd, mask=m_lt)
            plsc.store_scatter(aidx, [pos], gidx, mask=m_lt)
            m_eq = kd == T
            ceq = plsc.cumsum(m_eq.astype(jnp.int32))
            m_eq = jnp.logical_and(m_eq, (off_eq + ceq) <= need_eq)
            ceq2 = plsc.cumsum(m_eq.astype(jnp.int32))
            peq = count_lt + off_eq + ceq2 - 1
            plsc.store_scatter(fidx, [peq], gidx, mask=m_eq)
            off_lt = off_lt + jnp.sum(m_lt.astype(jnp.int32))
            off_eq = off_eq + jnp.sum(m_eq.astype(jnp.int32))
        return (off_lt, off_eq)

    lax.fori_loop(0, NV // UNROLL, comp, (jnp.int32(0), jnp.int32(0)))

    nv_lt = (count_lt + (L - 1)) // L
    bufs = [(akd, aidx), (bkd, bidx)]
    for p in range(4):
        sh = 8 * p
        skd, sidx = bufs[p % 2]
        dkd, didx = bufs[(p + 1) % 2]
        zero_hist()

        def hacc(i, c, _sh=sh, _skd=skd):
            m = (i * L + lanes) < count_lt
            kv = _skd[pl.ds(i * L, L)]
            d = srl(kv, _sh) & jnp.int32(255)
            plsc.addupdate_scatter(hist, [lanes * 256 + d], ones, mask=m)
            return c
        lax.fori_loop(0, nv_lt, hacc, 0)

        def offb(e, cin):
            tot = zeros
            for l in range(16):
                tot = tot + hist[pl.ds(l * 256 + e * L, L)]
            cum = plsc.cumsum(tot)
            offs[pl.ds(e * L, L)] = cin + cum - tot
            return cin + jnp.sum(tot)
        lax.fori_loop(0, 16, offb, jnp.int32(0))

        def scat(i, c, _p=p, _sh=sh, _skd=skd, _sidx=sidx, _dkd=dkd, _didx=didx):
            m = (i * L + lanes) < count_lt
            kv = _skd[pl.ds(i * L, L)]
            iv = _sidx[pl.ds(i * L, L)]
            d = srl(kv, _sh) & jnp.int32(255)
            base = plsc.load_gather(offs, [d])
            dup, lastm = plsc.scan_count(d, mask=m)
            pos = base + dup - 1
            if _p == 3:
                plsc.store_scatter(fidx, [pos], iv, mask=m)
            else:
                plsc.store_scatter(_dkd, [pos], kv, mask=m)
                plsc.store_scatter(_didx, [pos], iv, mask=m)
            plsc.addupdate_scatter(offs, [d], dup,
                                   mask=jnp.logical_and(lastm, m))
            return c
        lax.fori_loop(0, nv_lt, scat, 0)

    copies = []
    for i in range(K // L):
        idxv = fidx[pl.ds(i * L, L)]
        idxv = jnp.clip(idxv, 0, B * N - 1)
        copies.append(pltpu.async_copy(table_hbm.at[idxv],
                                       rows.at[pl.ds(i * L, L)], sem))
    for cp in copies:
        cp.wait()
    pltpu.sync_copy(rows, out_hbm.at[b])


def kernel(output_of_dgcnn_layer):
    xt = jnp.swapaxes(output_of_dgcnn_layer, 1, 2)
    x2d = output_of_dgcnn_layer.reshape(B * N, F)
    maxes = _compute_maxes(xt)
    return _sc_topk(maxes, x2d)


# speedup vs baseline: 1.2247x; 1.0020x over previous
import functools

import jax
import jax.numpy as jnp
from jax import lax
from jax.experimental import pallas as pl
from jax.experimental.pallas import tpu as pltpu
from jax.experimental.pallas import tpu_sc as plsc

B = 32
N = 32768
F = 64
K = 1024
L = 16
NV = N // L


def _max_body(xt_ref, o_ref):
    m = jnp.max(xt_ref[...], axis=1)
    o_ref[...] = jnp.where(m == 0.0, 0.0, m)


def _compute_maxes(xt):
    blk = 1024
    return pl.pallas_call(
        _max_body,
        grid=(N // blk,),
        in_specs=[pl.BlockSpec((B, F, blk), lambda i: (0, 0, i))],
        out_specs=pl.BlockSpec((B, blk), lambda i: (0, i)),
        out_shape=jax.ShapeDtypeStruct((B, N), jnp.float32),
    )(xt)


_sc_mesh = plsc.VectorSubcoreMesh(core_axis_name="c", subcore_axis_name="s")


@functools.partial(
    pl.kernel,
    out_type=jax.ShapeDtypeStruct((B, K, F), jnp.float32),
    mesh=_sc_mesh,
    compiler_params=pltpu.CompilerParams(needs_layout_passes=False,
                                         use_tc_tiling_on_sc=False),
    scratch_types=[
        pltpu.VMEM((N,), jnp.float32),
        pltpu.VMEM((4096,), jnp.int32),
        pltpu.VMEM((256,), jnp.int32),
        pltpu.VMEM((K,), jnp.int32),
        pltpu.VMEM((K,), jnp.int32),
        pltpu.VMEM((K,), jnp.int32),
        pltpu.VMEM((K,), jnp.int32),
        pltpu.VMEM((K,), jnp.int32),
        pltpu.VMEM((K, F), jnp.float32),
        pltpu.SemaphoreType.DMA,
    ],
)
def _sc_topk(maxes_hbm, table_hbm, out_hbm,
             maxv, hist, offs, akd, aidx, bkd, bidx, fidx, rows, sem):
    b = lax.axis_index("s") * 2 + lax.axis_index("c")
    pltpu.sync_copy(maxes_hbm.at[b], maxv)

    def zf(i, c):
        fidx[pl.ds(i * L, L)] = jnp.zeros((L,), jnp.int32)
        return c
    lax.fori_loop(0, K // L, zf, 0)

    lanes = lax.iota(jnp.int32, 16)
    ones = jnp.ones((L,), jnp.int32)
    zeros = jnp.zeros((L,), jnp.int32)

    def srl(x, s):
        return lax.shift_right_logical(
            x, jnp.full(jnp.shape(x), s, jnp.int32))

    def kd_at(i):
        return lax.bitcast_convert_type(maxv[pl.ds(i * L, L)], jnp.int32)

    def zero_hist():
        def zb(i, c):
            hist[pl.ds(i * L, L)] = zeros
            return c
        lax.fori_loop(0, 256, zb, 0)

    UNROLL = 4
    prefix = jnp.int32(0)
    cnt_before = jnp.int32(0)
    for p in range(4):
        sh = 24 - 8 * p
        zero_hist()

        def acc(i, c, _p=p, _sh=sh, _prefix=prefix):
            for u in range(UNROLL):
                base = (i * UNROLL + u) * L
                if _p == 0:
                    v = maxv[pl.ds(base, L)]
                    kb = lax.bitcast_convert_type(v, jnp.int32)
                    kd = jnp.where(kb < 0, kb, (~kb) & jnp.int32(0x7FFFFFFF))
                    maxv[pl.ds(base, L)] = lax.bitcast_convert_type(
                        kd, jnp.float32)
                    d = srl(kd, _sh) & jnp.int32(255)
                    plsc.addupdate_scatter(hist, [lanes * 256 + d], ones)
                else:
                    kd = lax.bitcast_convert_type(
                        maxv[pl.ds(base, L)], jnp.int32)
                    d = srl(kd, _sh) & jnp.int32(255)
                    m = srl(kd, _sh + 8) == srl(_prefix, _sh + 8)
                    plsc.addupdate_scatter(hist, [lanes * 256 + d], ones,
                                           mask=m)
            return c
        lax.fori_loop(0, NV // UNROLL, acc, 0)

        def chunk(e, carry):
            crun, tdig, cntb, done = carry
            tot = zeros
            for l in range(16):
                tot = tot + hist[pl.ds(l * 256 + e * L, L)]
            cum = plsc.cumsum(tot)
            reached = (crun + cum) >= K
            nbelow = jnp.sum(jnp.where(reached, 0, 1).astype(jnp.int32))
            below = jnp.sum(jnp.where(reached, 0, tot))
            found = nbelow < 16
            upd = jnp.logical_and(done == 0, found)
            tdig = jnp.where(upd, e * L + nbelow, tdig)
            cntb = jnp.where(upd, crun + below, cntb)
            crun = crun + jnp.sum(tot)
            done = jnp.where(upd, jnp.int32(1), done)
            return crun, tdig, cntb, done

        _, tdig, cntb, _ = lax.fori_loop(
            0, 16, chunk,
            (cnt_before, jnp.int32(0), jnp.int32(0), jnp.int32(0)))
        prefix = prefix | (tdig << sh)
        cnt_before = cntb

    T = prefix
    count_lt = cnt_before
    need_eq = K - count_lt
    MIN32 = jnp.int32(-2147483648)
    Tx = T ^ MIN32

    def comp(i, carry):
        off_lt, off_eq = carry
        for u in range(UNROLL):
            j = i * UNROLL + u
            kd = kd_at(j)
            gidx = j * L + lanes
            m_lt = (kd ^ MIN32) < Tx
            c = plsc.cumsum(m_lt.astype(jnp.int32))
            pos = off_lt + c - 1
            plsc.store_scatter(akd, [pos], kd, mask=m_lt)
            plsc.store_scatter(aidx, [pos], gidx, mask=m_lt)
            m_eq = kd == T
            ceq = plsc.cumsum(m_eq.astype(jnp.int32))
            m_eq = jnp.logical_and(m_eq, (off_eq + ceq) <= need_eq)
            ceq2 = plsc.cumsum(m_eq.astype(jnp.int32))
            peq = count_lt + off_eq + ceq2 - 1
            plsc.store_scatter(fidx, [peq], gidx, mask=m_eq)
            off_lt = off_lt + jnp.sum(m_lt.astype(jnp.int32))
            off_eq = off_eq + jnp.sum(m_eq.astype(jnp.int32))
        return (off_lt, off_eq)

    lax.fori_loop(0, NV // UNROLL, comp, (jnp.int32(0), jnp.int32(0)))

    nv_lt = (count_lt + (L - 1)) // L
    bufs = [(akd, aidx), (bkd, bidx)]
    for p in range(4):
        sh = 8 * p
        skd, sidx = bufs[p % 2]
        dkd, didx = bufs[(p + 1) % 2]
        zero_hist()

        def hacc(i, c, _sh=sh, _skd=skd):
            m = (i * L + lanes) < count_lt
            kv = _skd[pl.ds(i * L, L)]
            d = srl(kv, _sh) & jnp.int32(255)
            plsc.addupdate_scatter(hist, [lanes * 256 + d], ones, mask=m)
            return c
        lax.fori_loop(0, nv_lt, hacc, 0)

        def offb(e, cin):
            tot = zeros
            for l in range(16):
                tot = tot + hist[pl.ds(l * 256 + e * L, L)]
            cum = plsc.cumsum(tot)
            offs[pl.ds(e * L, L)] = cin + cum - tot
            return cin + jnp.sum(tot)
        lax.fori_loop(0, 16, offb, jnp.int32(0))

        def scat(i, c, _p=p, _sh=sh, _skd=skd, _sidx=sidx, _dkd=dkd, _didx=didx):
            m = (i * L + lanes) < count_lt
            kv = _skd[pl.ds(i * L, L)]
            iv = _sidx[pl.ds(i * L, L)]
            d = srl(kv, _sh) & jnp.int32(255)
            base = plsc.load_gather(offs, [d])
            dup, lastm = plsc.scan_count(d, mask=m)
            pos = base + dup - 1
            if _p == 3:
                plsc.store_scatter(fidx, [pos], iv, mask=m)
            else:
                plsc.store_scatter(_dkd, [pos], kv, mask=m)
                plsc.store_scatter(_didx, [pos], iv, mask=m)
            plsc.addupdate_scatter(offs, [d], dup,
                                   mask=jnp.logical_and(lastm, m))
            return c
        lax.fori_loop(0, nv_lt, scat, 0)

    copies = []
    for i in range(K // L):
        idxv = fidx[pl.ds(i * L, L)]
        idxv = jnp.clip(idxv, 0, N - 1)
        copies.append(pltpu.async_copy(table_hbm.at[b].at[idxv],
                                       rows.at[pl.ds(i * L, L)], sem))
    for cp in copies:
        cp.wait()
    pltpu.sync_copy(rows, out_hbm.at[b])


def kernel(output_of_dgcnn_layer):
    xt = jnp.swapaxes(output_of_dgcnn_layer, 1, 2)
    maxes = _compute_maxes(xt)
    return _sc_topk(maxes, output_of_dgcnn_layer)


# speedup vs baseline: 2.3263x; 1.8995x over previous
import functools

import jax
import jax.numpy as jnp
from jax import lax
from jax.experimental import pallas as pl
from jax.experimental.pallas import tpu as pltpu
from jax.experimental.pallas import tpu_sc as plsc

B = 32
N = 32768
F = 64
K = 1024
L = 16
NV = N // L


def _max_body(xt_ref, o_ref):
    m = jnp.max(xt_ref[...], axis=1)
    o_ref[...] = jnp.where(m == 0.0, 0.0, m)


def _compute_maxes(xt):
    blk = 1024
    return pl.pallas_call(
        _max_body,
        grid=(N // blk,),
        in_specs=[pl.BlockSpec((B, F, blk), lambda i: (0, 0, i))],
        out_specs=pl.BlockSpec((B, blk), lambda i: (0, i)),
        out_shape=jax.ShapeDtypeStruct((B, N), jnp.float32),
    )(xt)


_sc_mesh = plsc.VectorSubcoreMesh(core_axis_name="c", subcore_axis_name="s")


@functools.partial(
    pl.kernel,
    out_type=jax.ShapeDtypeStruct((B * K // 128, 128), jnp.int32),
    mesh=_sc_mesh,
    compiler_params=pltpu.CompilerParams(needs_layout_passes=False,
                                         use_tc_tiling_on_sc=False),
    scratch_types=[
        pltpu.VMEM((N,), jnp.float32),
        pltpu.VMEM((4096,), jnp.int32),
        pltpu.VMEM((256,), jnp.int32),
        pltpu.VMEM((K,), jnp.int32),
        pltpu.VMEM((K,), jnp.int32),
        pltpu.VMEM((K,), jnp.int32),
        pltpu.VMEM((K,), jnp.int32),
        pltpu.VMEM((8, 128), jnp.int32),
    ],
)
def _sc_select(maxes_hbm, fout_hbm,
               maxv, hist, offs, akd, aidx, bkd, bidx, fidx):
    b = lax.axis_index("s") * 2 + lax.axis_index("c")
    pltpu.sync_copy(maxes_hbm.at[b], maxv)

    def zf(i, c):
        fidx[i // 8, pl.ds((i % 8) * L, L)] = jnp.zeros((L,), jnp.int32)
        return c
    lax.fori_loop(0, K // L, zf, 0)

    lanes = lax.iota(jnp.int32, 16)
    ones = jnp.ones((L,), jnp.int32)
    zeros = jnp.zeros((L,), jnp.int32)

    def srl(x, s):
        return lax.shift_right_logical(
            x, jnp.full(jnp.shape(x), s, jnp.int32))

    def kd_at(i):
        return lax.bitcast_convert_type(maxv[pl.ds(i * L, L)], jnp.int32)

    def zero_hist():
        def zb(i, c):
            hist[pl.ds(i * L, L)] = zeros
            return c
        lax.fori_loop(0, 256, zb, 0)

    UNROLL = 4
    prefix = jnp.int32(0)
    cnt_before = jnp.int32(0)
    for p in range(4):
        sh = 24 - 8 * p
        zero_hist()

        def acc(i, c, _p=p, _sh=sh, _prefix=prefix):
            for u in range(UNROLL):
                base = (i * UNROLL + u) * L
                if _p == 0:
                    v = maxv[pl.ds(base, L)]
                    kb = lax.bitcast_convert_type(v, jnp.int32)
                    kd = jnp.where(kb < 0, kb, (~kb) & jnp.int32(0x7FFFFFFF))
                    maxv[pl.ds(base, L)] = lax.bitcast_convert_type(
                        kd, jnp.float32)
                    d = srl(kd, _sh) & jnp.int32(255)
                    plsc.addupdate_scatter(hist, [lanes * 256 + d], ones)
                else:
                    kd = lax.bitcast_convert_type(
                        maxv[pl.ds(base, L)], jnp.int32)
                    d = srl(kd, _sh) & jnp.int32(255)
                    m = srl(kd, _sh + 8) == srl(_prefix, _sh + 8)
                    plsc.addupdate_scatter(hist, [lanes * 256 + d], ones,
                                           mask=m)
            return c
        lax.fori_loop(0, NV // UNROLL, acc, 0)

        def chunk(e, carry):
            crun, tdig, cntb, done = carry
            tot = zeros
            for l in range(16):
                tot = tot + hist[pl.ds(l * 256 + e * L, L)]
            cum = plsc.cumsum(tot)
            reached = (crun + cum) >= K
            nbelow = jnp.sum(jnp.where(reached, 0, 1).astype(jnp.int32))
            below = jnp.sum(jnp.where(reached, 0, tot))
            found = nbelow < 16
            upd = jnp.logical_and(done == 0, found)
            tdig = jnp.where(upd, e * L + nbelow, tdig)
            cntb = jnp.where(upd, crun + below, cntb)
            crun = crun + jnp.sum(tot)
            done = jnp.where(upd, jnp.int32(1), done)
            return crun, tdig, cntb, done

        _, tdig, cntb, _ = lax.fori_loop(
            0, 16, chunk,
            (cnt_before, jnp.int32(0), jnp.int32(0), jnp.int32(0)))
        prefix = prefix | (tdig << sh)
        cnt_before = cntb

    T = prefix
    count_lt = cnt_before
    need_eq = K - count_lt
    MIN32 = jnp.int32(-2147483648)
    Tx = T ^ MIN32

    def comp(i, carry):
        off_lt, off_eq = carry
        for u in range(UNROLL):
            j = i * UNROLL + u
            kd = kd_at(j)
            gidx = j * L + lanes
            m_lt = (kd ^ MIN32) < Tx
            c = plsc.cumsum(m_lt.astype(jnp.int32))
            pos = off_lt + c - 1
            plsc.store_scatter(akd, [pos], kd, mask=m_lt)
            plsc.store_scatter(aidx, [pos], gidx, mask=m_lt)
            m_eq = kd == T
            ceq = plsc.cumsum(m_eq.astype(jnp.int32))
            m_eq = jnp.logical_and(m_eq, (off_eq + ceq) <= need_eq)
            ceq2 = plsc.cumsum(m_eq.astype(jnp.int32))
            peq = count_lt + off_eq + ceq2 - 1
            plsc.store_scatter(fidx, [peq >> 7, peq & 127], gidx, mask=m_eq)
            off_lt = off_lt + jnp.sum(m_lt.astype(jnp.int32))
            off_eq = off_eq + jnp.sum(m_eq.astype(jnp.int32))
        return (off_lt, off_eq)

    lax.fori_loop(0, NV // UNROLL, comp, (jnp.int32(0), jnp.int32(0)))

    nv_lt = (count_lt + (L - 1)) // L
    bufs = [(akd, aidx), (bkd, bidx)]
    for p in range(4):
        sh = 8 * p
        skd, sidx = bufs[p % 2]
        dkd, didx = bufs[(p + 1) % 2]
        zero_hist()

        def hacc(i, c, _sh=sh, _skd=skd):
            m = (i * L + lanes) < count_lt
            kv = _skd[pl.ds(i * L, L)]
            d = srl(kv, _sh) & jnp.int32(255)
            plsc.addupdate_scatter(hist, [lanes * 256 + d], ones, mask=m)
            return c
        lax.fori_loop(0, nv_lt, hacc, 0)

        def offb(e, cin):
            tot = zeros
            for l in range(16):
                tot = tot + hist[pl.ds(l * 256 + e * L, L)]
            cum = plsc.cumsum(tot)
            offs[pl.ds(e * L, L)] = cin + cum - tot
            return cin + jnp.sum(tot)
        lax.fori_loop(0, 16, offb, jnp.int32(0))

        def scat(i, c, _p=p, _sh=sh, _skd=skd, _sidx=sidx, _dkd=dkd, _didx=didx):
            m = (i * L + lanes) < count_lt
            kv = _skd[pl.ds(i * L, L)]
            iv = _sidx[pl.ds(i * L, L)]
            d = srl(kv, _sh) & jnp.int32(255)
            base = plsc.load_gather(offs, [d])
            dup, lastm = plsc.scan_count(d, mask=m)
            pos = base + dup - 1
            if _p == 3:
                plsc.store_scatter(fidx, [pos >> 7, pos & 127], iv, mask=m)
            else:
                plsc.store_scatter(_dkd, [pos], kv, mask=m)
                plsc.store_scatter(_didx, [pos], iv, mask=m)
            plsc.addupdate_scatter(offs, [d], dup,
                                   mask=jnp.logical_and(lastm, m))
            return c
        lax.fori_loop(0, nv_lt, scat, 0)

    pltpu.sync_copy(fidx, fout_hbm.at[pl.ds(b * 8, 8)])


CH = 512
NCH = N // CH


@functools.partial(
    pl.kernel,
    out_type=jax.ShapeDtypeStruct((B, K // 2, 128), jnp.float32),
    mesh=_sc_mesh,
    compiler_params=pltpu.CompilerParams(needs_layout_passes=False),
    scratch_types=[
        pltpu.VMEM((F, CH), jnp.float32),
        pltpu.VMEM((K // 2, 128), jnp.float32),
        pltpu.VMEM((8, 128), jnp.int32),
        pltpu.VMEM((K,), jnp.int32),
        pltpu.VMEM((K,), jnp.int32),
        pltpu.VMEM((NCH,), jnp.int32),
        pltpu.VMEM((NCH,), jnp.int32),
    ],
)
def _sc_extract(fidx_hbm, xt_hbm, out_hbm,
                chunk, rowbuf, fv, colb, rankb, ostart, orun):
    b = lax.axis_index("s") * 2 + lax.axis_index("c")
    lanes = lax.iota(jnp.int32, 16)
    pltpu.sync_copy(fidx_hbm.at[pl.ds(b * 8, 8)], fv)

    for g in range(NCH // L):
        orun[pl.ds(g * L, L)] = jnp.zeros((L,), jnp.int32)

    def hacc(i, c):
        cols = fv[i // 8, pl.ds((i % 8) * L, L)]
        ch = cols // CH
        dup, last = plsc.scan_count(ch)
        plsc.addupdate_scatter(orun, [ch], dup, mask=last)
        return c
    lax.fori_loop(0, K // L, hacc, 0)

    carry = jnp.int32(0)
    for g in range(NCH // L):
        v = orun[pl.ds(g * L, L)]
        cum = plsc.cumsum(v)
        excl = carry + cum - v
        ostart[pl.ds(g * L, L)] = excl
        orun[pl.ds(g * L, L)] = excl
        carry = carry + jnp.sum(v)

    def bkt(i, c):
        cols = fv[i // 8, pl.ds((i % 8) * L, L)]
        rank = i * L + lanes
        ch = cols // CH
        base = plsc.load_gather(orun, [ch])
        dup, last = plsc.scan_count(ch)
        pos = base + dup - 1
        plsc.store_scatter(colb, [pos], cols - ch * CH)
        plsc.store_scatter(rankb, [pos], rank)
        plsc.addupdate_scatter(orun, [ch], dup, mask=last)
        return c
    lax.fori_loop(0, K // L, bkt, 0)

    def one_chunk(ci, c):
        pltpu.sync_copy(xt_hbm.at[b, :, pl.ds(ci * CH, CH)], chunk)
        sv = ostart[pl.ds((ci // L) * L, L)]
        ev = orun[pl.ds((ci // L) * L, L)]
        lsel = (lanes == (ci % L))
        start = jnp.sum(jnp.where(lsel, sv, 0))
        end = jnp.sum(jnp.where(lsel, ev, 0))

        def grp(g, c2):
            idx = start + g * L + lanes
            m = idx < end
            cols16 = plsc.load_gather(colb, [idx], mask=m)
            rk = plsc.load_gather(rankb, [idx], mask=m)
            rrow = rk >> 1
            rcol0 = (rk & 1) * F
            for f in range(F):
                vals = plsc.load_gather(
                    chunk, [jnp.full((L,), f, jnp.int32), cols16], mask=m)
                plsc.store_scatter(rowbuf, [rrow, rcol0 + f], vals, mask=m)
            return c2
        lax.fori_loop(0, (end - start + (L - 1)) // L, grp, 0)
        return c
    lax.fori_loop(0, NCH, one_chunk, 0)

    pltpu.sync_copy(rowbuf, out_hbm.at[b])


def kernel(output_of_dgcnn_layer):
    xt = jnp.swapaxes(output_of_dgcnn_layer, 1, 2)
    maxes = _compute_maxes(xt)
    fidx = _sc_select(maxes)
    rows = _sc_extract(fidx, xt)
    return rows.reshape(B, K, F)


# speedup vs baseline: 2.3667x; 1.0174x over previous
import functools

import jax
import jax.numpy as jnp
from jax import lax
from jax.experimental import pallas as pl
from jax.experimental.pallas import tpu as pltpu
from jax.experimental.pallas import tpu_sc as plsc

B = 32
N = 32768
F = 64
K = 1024
L = 16
NV = N // L


def _max_body(xt_ref, o_ref):
    m = jnp.max(xt_ref[...], axis=1)
    o_ref[...] = jnp.where(m == 0.0, 0.0, m)


def _compute_maxes(xt):
    blk = 1024
    return pl.pallas_call(
        _max_body,
        grid=(N // blk,),
        in_specs=[pl.BlockSpec((B, F, blk), lambda i: (0, 0, i))],
        out_specs=pl.BlockSpec((B, blk), lambda i: (0, i)),
        out_shape=jax.ShapeDtypeStruct((B, N), jnp.float32),
    )(xt)


_sc_mesh = plsc.VectorSubcoreMesh(core_axis_name="c", subcore_axis_name="s")


@functools.partial(
    pl.kernel,
    out_type=jax.ShapeDtypeStruct((B * K // 128, 128), jnp.int32),
    mesh=_sc_mesh,
    compiler_params=pltpu.CompilerParams(needs_layout_passes=False,
                                         use_tc_tiling_on_sc=False),
    scratch_types=[
        pltpu.VMEM((N,), jnp.float32),
        pltpu.VMEM((4096,), jnp.int32),
        pltpu.VMEM((256,), jnp.int32),
        pltpu.VMEM((K,), jnp.int32),
        pltpu.VMEM((K,), jnp.int32),
        pltpu.VMEM((K,), jnp.int32),
        pltpu.VMEM((K,), jnp.int32),
        pltpu.VMEM((8, 128), jnp.int32),
    ],
)
def _sc_select(maxes_hbm, fout_hbm,
               maxv, hist, offs, akd, aidx, bkd, bidx, fidx):
    b = lax.axis_index("s") * 2 + lax.axis_index("c")
    pltpu.sync_copy(maxes_hbm.at[b], maxv)

    def zf(i, c):
        fidx[i // 8, pl.ds((i % 8) * L, L)] = jnp.zeros((L,), jnp.int32)
        return c
    lax.fori_loop(0, K // L, zf, 0)

    lanes = lax.iota(jnp.int32, 16)
    ones = jnp.ones((L,), jnp.int32)
    zeros = jnp.zeros((L,), jnp.int32)

    def srl(x, s):
        return lax.shift_right_logical(
            x, jnp.full(jnp.shape(x), s, jnp.int32))

    def kd_at(i):
        return lax.bitcast_convert_type(maxv[pl.ds(i * L, L)], jnp.int32)

    def zero_hist():
        def zb(i, c):
            hist[pl.ds(i * L, L)] = zeros
            return c
        lax.fori_loop(0, 256, zb, 0)

    UNROLL = 4
    prefix = jnp.int32(0)
    cnt_before = jnp.int32(0)
    for p in range(4):
        sh = 24 - 8 * p
        zero_hist()

        def acc(i, c, _p=p, _sh=sh, _prefix=prefix):
            for u in range(UNROLL):
                base = (i * UNROLL + u) * L
                if _p == 0:
                    v = maxv[pl.ds(base, L)]
                    kb = lax.bitcast_convert_type(v, jnp.int32)
                    kd = jnp.where(kb < 0, kb, (~kb) & jnp.int32(0x7FFFFFFF))
                    maxv[pl.ds(base, L)] = lax.bitcast_convert_type(
                        kd, jnp.float32)
                    d = srl(kd, _sh) & jnp.int32(255)
                    plsc.addupdate_scatter(hist, [lanes * 256 + d], ones)
                else:
                    kd = lax.bitcast_convert_type(
                        maxv[pl.ds(base, L)], jnp.int32)
                    d = srl(kd, _sh) & jnp.int32(255)
                    m = srl(kd, _sh + 8) == srl(_prefix, _sh + 8)
                    plsc.addupdate_scatter(hist, [lanes * 256 + d], ones,
                                           mask=m)
            return c
        lax.fori_loop(0, NV // UNROLL, acc, 0)

        def chunk(e, carry):
            crun, tdig, cntb, done = carry
            tot = zeros
            for l in range(16):
                tot = tot + hist[pl.ds(l * 256 + e * L, L)]
            cum = plsc.cumsum(tot)
            reached = (crun + cum) >= K
            nbelow = jnp.sum(jnp.where(reached, 0, 1).astype(jnp.int32))
            below = jnp.sum(jnp.where(reached, 0, tot))
            found = nbelow < 16
            upd = jnp.logical_and(done == 0, found)
            tdig = jnp.where(upd, e * L + nbelow, tdig)
            cntb = jnp.where(upd, crun + below, cntb)
            crun = crun + jnp.sum(tot)
            done = jnp.where(upd, jnp.int32(1), done)
            return crun, tdig, cntb, done

        _, tdig, cntb, _ = lax.fori_loop(
            0, 16, chunk,
            (cnt_before, jnp.int32(0), jnp.int32(0), jnp.int32(0)))
        prefix = prefix | (tdig << sh)
        cnt_before = cntb

    T = prefix
    count_lt = cnt_before
    need_eq = K - count_lt
    MIN32 = jnp.int32(-2147483648)
    Tx = T ^ MIN32

    def comp(i, carry):
        off_lt, off_eq = carry
        for u in range(UNROLL):
            j = i * UNROLL + u
            kd = kd_at(j)
            gidx = j * L + lanes
            m_lt = (kd ^ MIN32) < Tx
            c = plsc.cumsum(m_lt.astype(jnp.int32))
            pos = off_lt + c - 1
            plsc.store_scatter(akd, [pos], kd, mask=m_lt)
            plsc.store_scatter(aidx, [pos], gidx, mask=m_lt)
            n_lt = plsc.all_reduce_population_count(m_lt)
            m_eq = kd == T
            ceq = plsc.cumsum(m_eq.astype(jnp.int32))
            m_eq = jnp.logical_and(m_eq, (off_eq + ceq) <= need_eq)
            peq = count_lt + off_eq + ceq - 1
            plsc.store_scatter(fidx, [peq >> 7, peq & 127], gidx, mask=m_eq)
            n_eq = plsc.all_reduce_population_count(m_eq)
            off_lt = off_lt + n_lt
            off_eq = off_eq + n_eq
        return (off_lt, off_eq)

    lax.fori_loop(0, NV // UNROLL, comp,
                  (jnp.zeros((L,), jnp.int32), jnp.zeros((L,), jnp.int32)))

    nv_lt = (count_lt + (L - 1)) // L
    bufs = [(akd, aidx), (bkd, bidx)]
    for p in range(4):
        sh = 8 * p
        skd, sidx = bufs[p % 2]
        dkd, didx = bufs[(p + 1) % 2]
        zero_hist()

        def hacc(i, c, _sh=sh, _skd=skd):
            m = (i * L + lanes) < count_lt
            kv = _skd[pl.ds(i * L, L)]
            d = srl(kv, _sh) & jnp.int32(255)
            plsc.addupdate_scatter(hist, [lanes * 256 + d], ones, mask=m)
            return c
        lax.fori_loop(0, nv_lt, hacc, 0)

        def offb(e, cin):
            tot = zeros
            for l in range(16):
                tot = tot + hist[pl.ds(l * 256 + e * L, L)]
            cum = plsc.cumsum(tot)
            offs[pl.ds(e * L, L)] = cin + cum - tot
            return cin + jnp.sum(tot)
        lax.fori_loop(0, 16, offb, jnp.int32(0))

        def scat(i, c, _p=p, _sh=sh, _skd=skd, _sidx=sidx, _dkd=dkd, _didx=didx):
            m = (i * L + lanes) < count_lt
            kv = _skd[pl.ds(i * L, L)]
            iv = _sidx[pl.ds(i * L, L)]
            d = srl(kv, _sh) & jnp.int32(255)
            base = plsc.load_gather(offs, [d])
            dup, lastm = plsc.scan_count(d, mask=m)
            pos = base + dup - 1
            if _p == 3:
                plsc.store_scatter(fidx, [pos >> 7, pos & 127], iv, mask=m)
            else:
                plsc.store_scatter(_dkd, [pos], kv, mask=m)
                plsc.store_scatter(_didx, [pos], iv, mask=m)
            plsc.addupdate_scatter(offs, [d], dup,
                                   mask=jnp.logical_and(lastm, m))
            return c
        lax.fori_loop(0, nv_lt, scat, 0)

    pltpu.sync_copy(fidx, fout_hbm.at[pl.ds(b * 8, 8)])


CH = 512
NCH = N // CH


@functools.partial(
    pl.kernel,
    out_type=jax.ShapeDtypeStruct((B, K // 2, 128), jnp.float32),
    mesh=_sc_mesh,
    compiler_params=pltpu.CompilerParams(needs_layout_passes=False),
    scratch_types=[
        pltpu.VMEM((F, CH), jnp.float32),
        pltpu.VMEM((K // 2, 128), jnp.float32),
        pltpu.VMEM((8, 128), jnp.int32),
        pltpu.VMEM((K,), jnp.int32),
        pltpu.VMEM((K,), jnp.int32),
        pltpu.VMEM((NCH,), jnp.int32),
        pltpu.VMEM((NCH,), jnp.int32),
    ],
)
def _sc_extract(fidx_hbm, xt_hbm, out_hbm,
                chunk, rowbuf, fv, colb, rankb, ostart, orun):
    b = lax.axis_index("s") * 2 + lax.axis_index("c")
    lanes = lax.iota(jnp.int32, 16)
    pltpu.sync_copy(fidx_hbm.at[pl.ds(b * 8, 8)], fv)

    for g in range(NCH // L):
        orun[pl.ds(g * L, L)] = jnp.zeros((L,), jnp.int32)

    def hacc(i, c):
        cols = fv[i // 8, pl.ds((i % 8) * L, L)]
        ch = cols // CH
        dup, last = plsc.scan_count(ch)
        plsc.addupdate_scatter(orun, [ch], dup, mask=last)
        return c
    lax.fori_loop(0, K // L, hacc, 0)

    carry = jnp.int32(0)
    for g in range(NCH // L):
        v = orun[pl.ds(g * L, L)]
        cum = plsc.cumsum(v)
        excl = carry + cum - v
        ostart[pl.ds(g * L, L)] = excl
        orun[pl.ds(g * L, L)] = excl
        carry = carry + jnp.sum(v)

    def bkt(i, c):
        cols = fv[i // 8, pl.ds((i % 8) * L, L)]
        rank = i * L + lanes
        ch = cols // CH
        base = plsc.load_gather(orun, [ch])
        dup, last = plsc.scan_count(ch)
        pos = base + dup - 1
        plsc.store_scatter(colb, [pos], cols - ch * CH)
        plsc.store_scatter(rankb, [pos], rank)
        plsc.addupdate_scatter(orun, [ch], dup, mask=last)
        return c
    lax.fori_loop(0, K // L, bkt, 0)

    def one_chunk(ci, c):
        pltpu.sync_copy(xt_hbm.at[b, :, pl.ds(ci * CH, CH)], chunk)
        sv = ostart[pl.ds((ci // L) * L, L)]
        ev = orun[pl.ds((ci // L) * L, L)]
        lsel = (lanes == (ci % L))
        start = jnp.sum(jnp.where(lsel, sv, 0))
        end = jnp.sum(jnp.where(lsel, ev, 0))

        def grp(g, c2):
            idx = start + g * L + lanes
            m = idx < end
            cols16 = plsc.load_gather(colb, [idx], mask=m)
            rk = plsc.load_gather(rankb, [idx], mask=m)
            rrow = rk >> 1
            rcol0 = (rk & 1) * F
            for f in range(F):
                vals = plsc.load_gather(
                    chunk, [jnp.full((L,), f, jnp.int32), cols16], mask=m)
                plsc.store_scatter(rowbuf, [rrow, rcol0 + f], vals, mask=m)
            return c2
        lax.fori_loop(0, (end - start + (L - 1)) // L, grp, 0)
        return c
    lax.fori_loop(0, NCH, one_chunk, 0)

    pltpu.sync_copy(rowbuf, out_hbm.at[b])


def kernel(output_of_dgcnn_layer):
    xt = jnp.swapaxes(output_of_dgcnn_layer, 1, 2)
    maxes = _compute_maxes(xt)
    fidx = _sc_select(maxes)
    rows = _sc_extract(fidx, xt)
    return rows.reshape(B, K, F)
